# Initial kernel scaffold; baseline (speedup 1.0000x reference)
import jax
import jax.numpy as jnp
from jax.experimental import pallas as pl


def kernel(h, edge_index, ntype, etype, k_weight, q_weight, v_weight, a_weight, relation_pri, relation_att, relation_msg, skip):
    raise NotImplementedError("write your pallas kernel here")


# speedup vs baseline: 10.0779x; 10.0779x over previous
import functools
import math

import jax
import jax.numpy as jnp
from jax import lax
from jax.experimental import pallas as pl
from jax.experimental.pallas import tpu as pltpu
from jax.experimental.pallas import tpu_sc as plsc

_SC_PARAMS = pltpu.CompilerParams(
    use_tc_tiling_on_sc=False, needs_layout_passes=False)

N_NODES = 10000
N_EDGES = 160000
IN_DIM = 128
OUT_DIM = 64
N_NTYPES = 8
N_RELS = 8
N_HEADS = 8
D_K = 8

NA = 10048
NW = 32
CHUNK = 256
CPW = 20
E_PAD = NW * CPW * CHUNK
NPAIR = CPW // 2
ROWS_PER_SUB = NA // 16
FLUSH_HALF = ROWS_PER_SUB // 2


_TCB = 1256


def _tc0_body(h_ref, nt_ref, wcat_ref, bacat_ref, bmcat_ref,
              q_ref, k2_ref, v2_ref):
    h = h_ref[...]
    nt = nt_ref[...]
    p = jnp.dot(h, wcat_ref[...], preferred_element_type=jnp.float32)
    kn = jnp.zeros((_TCB, OUT_DIM), jnp.float32)
    qn = jnp.zeros((_TCB, OUT_DIM), jnp.float32)
    vn = jnp.zeros((_TCB, OUT_DIM), jnp.float32)
    for t in range(N_NTYPES):
        m = (nt == t)
        kn = kn + jnp.where(m, p[:, t * 64:(t + 1) * 64], 0.0)
        qn = qn + jnp.where(m, p[:, 512 + t * 64:512 + (t + 1) * 64], 0.0)
        vn = vn + jnp.where(m, p[:, 1024 + t * 64:1024 + (t + 1) * 64], 0.0)
    q_ref[...] = qn
    k2all = jnp.dot(kn, bacat_ref[...], preferred_element_type=jnp.float32)
    v2all = jnp.dot(vn, bmcat_ref[...], preferred_element_type=jnp.float32)
    for r in range(N_RELS):
        k2_ref[r] = k2all[:, r * 64:(r + 1) * 64]
        v2_ref[r] = v2all[:, r * 64:(r + 1) * 64]


def _tc0(h_pad, nt_pad, wcat, bacat, bmcat):
    grid = NA // _TCB
    return pl.pallas_call(
        _tc0_body,
        grid=(grid,),
        in_specs=[
            pl.BlockSpec((_TCB, IN_DIM), lambda i: (i, 0)),
            pl.BlockSpec((_TCB, 1), lambda i: (i, 0)),
            pl.BlockSpec((IN_DIM, 1536), lambda i: (0, 0)),
            pl.BlockSpec((OUT_DIM, 512), lambda i: (0, 0)),
            pl.BlockSpec((OUT_DIM, 512), lambda i: (0, 0)),
        ],
        out_specs=[
            pl.BlockSpec((_TCB, OUT_DIM), lambda i: (i, 0)),
            pl.BlockSpec((N_RELS, _TCB, OUT_DIM), lambda i: (0, i, 0)),
            pl.BlockSpec((N_RELS, _TCB, OUT_DIM), lambda i: (0, i, 0)),
        ],
        out_shape=[
            jax.ShapeDtypeStruct((NA, OUT_DIM), jnp.float32),
            jax.ShapeDtypeStruct((N_RELS, NA, OUT_DIM), jnp.float32),
            jax.ShapeDtypeStruct((N_RELS, NA, OUT_DIM), jnp.float32),
        ],
    )(h_pad, nt_pad, wcat, bacat, bmcat)


def _iota16():
    return lax.iota(jnp.int32, 16)


def _idx_issue(edge2_hbms, row0, bufs, sem, nsub=2):
    for hbm, buf in zip(edge2_hbms, bufs):
        pltpu.async_copy(hbm.at[pl.ds(row0, nsub)], buf, sem)


def _idx_wait(edge2_hbms, row0, bufs, sem, nsub=2):
    for hbm, buf in zip(edge2_hbms, bufs):
        pltpu.make_async_copy(hbm.at[pl.ds(row0, nsub)], buf, sem).wait()


def _kidx_compute(etv, srcv, kidxv, nsub=2):
    for j in range(nsub):
        for l in range(8):
            sl = pl.ds(l * 16, 16)
            kidxv[j, sl] = etv[j, sl] * NA + srcv[j, sl]


def _gather_issue(table_hbm, idxv, rows, sem, nsub=2):
    for j in range(nsub):
        pltpu.async_copy(table_hbm.at[idxv.at[j]],
                         rows.at[pl.ds(j * 128, 128)], sem)


def _gather_wait(table_hbm, idxv, rows, sem, nsub=2):
    for j in range(nsub):
        pltpu.make_async_copy(table_hbm.at[idxv.at[j]],
                              rows.at[pl.ds(j * 128, 128)], sem).wait()


def _scatter_add_issue(datav, shared, idxv, sem, nsub=2):
    for j in range(nsub):
        pltpu.async_copy(datav.at[pl.ds(j * 128, 128)],
                         shared.at[idxv.at[j]], sem, add=True)


def _scatter_add_wait(datav, shared, idxv, sem, nsub=2):
    for j in range(nsub):
        pltpu.make_async_copy(datav.at[pl.ds(j * 128, 128)],
                              shared.at[idxv.at[j]], sem).wait()


def _sca_body(k2_hbm, qn_hbm, src_hbm, dst_hbm, et_hbm,
              attn_hbm, tmax_hbm,
              srcv, dstv, etv, kidxv, k2rows, qrows, attnv, mscr, semi, semg,
              semw):
    c = lax.axis_index("c")
    s = lax.axis_index("s")
    wid = s * 2 + c
    iota = _iota16()
    eids = [iota + g * 16 for g in range(16)]
    base = wid * CPW

    def idx_row(ch):
        return 2 * (base + ch)

    def e0(ch):
        return (base + ch) * CHUNK

    edge_hbms = (src_hbm, dst_hbm, et_hbm)

    def bufs(sl):
        return (srcv.at[sl], dstv.at[sl], etv.at[sl])

    def compute_attn(ch, sl, mx):
        def head(h, mx):
            accs = [jnp.zeros((16,), jnp.float32) for _ in range(16)]
            for d in range(D_K):
                col = jnp.full((16,), h * 8 + d, jnp.int32)
                for g in range(16):
                    k16 = plsc.load_gather(k2rows.at[sl], [eids[g], col])
                    q16 = plsc.load_gather(qrows.at[sl], [eids[g], col])
                    accs[g] = accs[g] + k16 * q16
            hcol = jnp.full((16,), h, jnp.int32)
            for g in range(16):
                plsc.store_scatter(attnv.at[sl], [eids[g], hcol], accs[g])
                mx = jnp.maximum(mx, jnp.abs(accs[g]))
            return mx
        return pl.loop(0, N_HEADS, init_carry=mx)(head)

    _idx_issue(edge_hbms, idx_row(0), bufs(0), semi[0])
    _idx_issue(edge_hbms, idx_row(1), bufs(1), semi[1])
    _idx_wait(edge_hbms, idx_row(0), bufs(0), semi[0])
    _kidx_compute(etv.at[0], srcv.at[0], kidxv.at[0])
    _gather_issue(k2_hbm, kidxv.at[0], k2rows.at[0], semg[0])
    _gather_issue(qn_hbm, dstv.at[0], qrows.at[0], semg[0])

    def pair(p, mx):
        a = 2 * p
        b = 2 * p + 1

        def half(ch, sl, osl, mx):
            @pl.when(ch + 1 < CPW)
            def _():
                _idx_wait(edge_hbms, idx_row(ch + 1), bufs(osl), semi[osl])
                _kidx_compute(etv.at[osl], srcv.at[osl], kidxv.at[osl])
                _gather_issue(k2_hbm, kidxv.at[osl], k2rows.at[osl],
                              semg[osl])
                _gather_issue(qn_hbm, dstv.at[osl], qrows.at[osl], semg[osl])

            _gather_wait(k2_hbm, kidxv.at[sl], k2rows.at[sl], semg[sl])
            _gather_wait(qn_hbm, dstv.at[sl], qrows.at[sl], semg[sl])

            @pl.when(ch + 2 < CPW)
            def _():
                _idx_issue(edge_hbms, idx_row(ch + 2), bufs(sl), semi[sl])

            @pl.when(ch >= 2)
            def _():
                pltpu.make_async_copy(
                    attnv.at[sl], attn_hbm.at[pl.ds(e0(ch - 2), CHUNK)],
                    semw[sl]).wait()

            mx = compute_attn(ch, sl, mx)
            pltpu.async_copy(attnv.at[sl], attn_hbm.at[pl.ds(e0(ch), CHUNK)],
                             semw[sl])
            return mx

        mx = half(a, 0, 1, mx)
        mx = half(b, 1, 0, mx)
        return mx

    mx = pl.loop(0, NPAIR, init_carry=jnp.zeros((16,), jnp.float32))(pair)
    pltpu.make_async_copy(attnv.at[0],
                          attn_hbm.at[pl.ds(e0(CPW - 2), CHUNK)],
                          semw[0]).wait()
    pltpu.make_async_copy(attnv.at[1],
                          attn_hbm.at[pl.ds(e0(CPW - 1), CHUNK)],
                          semw[1]).wait()
    mscr[...] = mx
    pltpu.sync_copy(mscr, tmax_hbm.at[wid])


def _sca(k2f, qn, src2, dst2, et2):
    mesh = plsc.VectorSubcoreMesh(core_axis_name="c", subcore_axis_name="s")
    f = pl.kernel(
        _sca_body,
        out_type=[
            jax.ShapeDtypeStruct((E_PAD, N_HEADS), jnp.float32),
            jax.ShapeDtypeStruct((NW, 16), jnp.float32),
        ],
        mesh=mesh,
        compiler_params=_SC_PARAMS,
        scratch_types=[
            pltpu.VMEM((2, 2, 128), jnp.int32),
            pltpu.VMEM((2, 2, 128), jnp.int32),
            pltpu.VMEM((2, 2, 128), jnp.int32),
            pltpu.VMEM((2, 2, 128), jnp.int32),
            pltpu.VMEM((2, CHUNK, OUT_DIM), jnp.float32),
            pltpu.VMEM((2, CHUNK, OUT_DIM), jnp.float32),
            pltpu.VMEM((2, CHUNK, N_HEADS), jnp.float32),
            pltpu.VMEM((16,), jnp.float32),
            [pltpu.SemaphoreType.DMA, pltpu.SemaphoreType.DMA],
            [pltpu.SemaphoreType.DMA, pltpu.SemaphoreType.DMA],
            [pltpu.SemaphoreType.DMA, pltpu.SemaphoreType.DMA],
        ],
    )
    return f(k2f, qn, src2, dst2, et2)


def _copy_idx(src_ref, dst_ref, nsub=2):
    for j in range(nsub):
        for l in range(8):
            sl = pl.ds(l * 16, 16)
            dst_ref[j, sl] = src_ref[j, sl]


def _scb_body(attn_hbm, dst_hbm, invlam_hbm, z8_hbm,
              s1out_hbm,
              dstv, dscv, attnv, edv, lamv, flushv, s1_sh, semi, sems):
    c = lax.axis_index("c")
    s = lax.axis_index("s")
    wid = s * 2 + c
    iota = _iota16()
    idiv = lax.shift_right_logical(iota, 3)
    imod = lax.bitwise_and(iota, jnp.full((16,), 7, jnp.int32))
    base = wid * CPW

    r0 = s * ROWS_PER_SUB
    pltpu.sync_copy(z8_hbm.at[pl.ds(r0, ROWS_PER_SUB)],
                    s1_sh.at[pl.ds(r0, ROWS_PER_SUB)])
    pltpu.sync_copy(invlam_hbm, lamv)
    plsc.subcore_barrier()
    inv = lamv[...]

    def in_issue(ch, sl):
        pltpu.async_copy(dst_hbm.at[pl.ds(2 * (base + ch), 2)], dstv.at[sl],
                         semi[sl])
        pltpu.async_copy(attn_hbm.at[pl.ds((base + ch) * CHUNK, CHUNK)],
                         attnv.at[sl], semi[sl])

    def in_wait(ch, sl):
        pltpu.make_async_copy(dst_hbm.at[pl.ds(2 * (base + ch), 2)],
                              dstv.at[sl], semi[sl]).wait()
        pltpu.make_async_copy(attn_hbm.at[pl.ds((base + ch) * CHUNK, CHUNK)],
                              attnv.at[sl], semi[sl]).wait()

    in_issue(0, 0)
    in_issue(1, 1)

    def half(ch, sl):
        in_wait(ch, sl)

        @pl.when(ch >= 2)
        def _():
            _scatter_add_wait(edv.at[sl], s1_sh, dscv.at[sl], sems[sl])

        _copy_idx(dstv.at[sl], dscv.at[sl])

        def block(j, _):
            row = idiv + 2 * j
            a16 = plsc.load_gather(attnv.at[sl], [row, imod])
            e16 = jnp.exp(a16 * inv)
            plsc.store_scatter(edv.at[sl], [row, imod], e16)
            return 0

        pl.loop(0, CHUNK * N_HEADS // 16, init_carry=0)(block)
        _scatter_add_issue(edv.at[sl], s1_sh, dscv.at[sl], sems[sl])

        @pl.when(ch + 2 < CPW)
        def _():
            in_issue(ch + 2, sl)

    def pair(p, _):
        half(2 * p, 0)
        half(2 * p + 1, 1)
        return 0

    pl.loop(0, NPAIR, init_carry=0)(pair)
    _scatter_add_wait(edv.at[0], s1_sh, dscv.at[0], sems[0])
    _scatter_add_wait(edv.at[1], s1_sh, dscv.at[1], sems[1])
    plsc.subcore_barrier()
    pltpu.sync_copy(s1_sh.at[pl.ds(r0, ROWS_PER_SUB)], flushv)
    pltpu.sync_copy(flushv, s1out_hbm.at[c, pl.ds(r0, ROWS_PER_SUB)])


def _scb(attn, dst2, invlam, z8):
    mesh = plsc.VectorSubcoreMesh(core_axis_name="c", subcore_axis_name="s")
    f = pl.kernel(
        _scb_body,
        out_type=[jax.ShapeDtypeStruct((2, NA, N_HEADS), jnp.float32)],
        mesh=mesh,
        compiler_params=_SC_PARAMS,
        scratch_types=[
            pltpu.VMEM((2, 2, 128), jnp.int32),
            pltpu.VMEM((2, 2, 128), jnp.int32),
            pltpu.VMEM((2, CHUNK, N_HEADS), jnp.float32),
            pltpu.VMEM((2, CHUNK, N_HEADS), jnp.float32),
            pltpu.VMEM((16,), jnp.float32),
            pltpu.VMEM((ROWS_PER_SUB, N_HEADS), jnp.float32),
            pltpu.VMEM_SHARED((NA, N_HEADS), jnp.float32),
            [pltpu.SemaphoreType.DMA, pltpu.SemaphoreType.DMA],
            [pltpu.SemaphoreType.DMA, pltpu.SemaphoreType.DMA],
        ],
    )
    return f(attn, dst2, invlam, z8)[0]


def _tc1_body(s1_ref, lam_ref, m_ref):
    s1 = s1_ref[0] + s1_ref[1]
    m_ref[...] = lam_ref[0, 0] * jnp.log(s1)


def _tc1(s1p, lam11):
    return pl.pallas_call(
        _tc1_body,
        out_shape=jax.ShapeDtypeStruct((NA, N_HEADS), jnp.float32),
    )(s1p, lam11)


_CH_C = 128
_CPW_C = E_PAD // (NW * _CH_C)
_NPAIR_C = _CPW_C // 2


def _scc_body(attn_hbm, src_hbm, dst_hbm, et_hbm, m_hbm, v2_hbm, z8_hbm,
              z64_hbm,
              den_out, tn_out,
              srcv, dstv, dscv, etv, kidxv, attnv, mrows, v2rows, edv, wvv,
              den_sh, tn_sh, semi, semg, semd, semt):
    c = lax.axis_index("c")
    s = lax.axis_index("s")
    wid = s * 2 + c
    iota = _iota16()
    idiv = lax.shift_right_logical(iota, 3)
    imod = lax.bitwise_and(iota, jnp.full((16,), 7, jnp.int32))
    eids = [iota + g * 16 for g in range(8)]
    base = wid * _CPW_C

    r0 = s * ROWS_PER_SUB
    pltpu.sync_copy(z8_hbm.at[pl.ds(r0, ROWS_PER_SUB)],
                    den_sh.at[pl.ds(r0, ROWS_PER_SUB)])
    pltpu.sync_copy(z64_hbm.at[pl.ds(r0, ROWS_PER_SUB)],
                    tn_sh.at[pl.ds(r0, ROWS_PER_SUB)])
    plsc.subcore_barrier()

    edge_hbms = (src_hbm, dst_hbm, et_hbm)

    def bufs(sl):
        return (srcv.at[sl], dstv.at[sl], etv.at[sl])

    def in_issue(ch, sl):
        _idx_issue(edge_hbms, base + ch, bufs(sl), semi[sl], nsub=1)
        pltpu.async_copy(attn_hbm.at[pl.ds((base + ch) * _CH_C, _CH_C)],
                         attnv.at[sl], semi[sl])

    def in_wait(ch, sl):
        _idx_wait(edge_hbms, base + ch, bufs(sl), semi[sl], nsub=1)
        pltpu.make_async_copy(attn_hbm.at[pl.ds((base + ch) * _CH_C, _CH_C)],
                              attnv.at[sl], semi[sl]).wait()

    def gathers_issue(sl):
        _gather_issue(v2_hbm, kidxv.at[sl], v2rows.at[sl], semg[sl], nsub=1)
        _gather_issue(m_hbm, dstv.at[sl], mrows.at[sl], semg[sl], nsub=1)

    def gathers_wait(sl):
        _gather_wait(v2_hbm, kidxv.at[sl], v2rows.at[sl], semg[sl], nsub=1)
        _gather_wait(m_hbm, dstv.at[sl], mrows.at[sl], semg[sl], nsub=1)

    in_issue(0, 0)
    in_issue(1, 1)
    in_wait(0, 0)
    _kidx_compute(etv.at[0], srcv.at[0], kidxv.at[0], nsub=1)
    gathers_issue(0)

    def half(ch, sl, osl):
        @pl.when(ch + 1 < _CPW_C)
        def _():
            in_wait(ch + 1, osl)
            _kidx_compute(etv.at[osl], srcv.at[osl], kidxv.at[osl], nsub=1)
            gathers_issue(osl)

        gathers_wait(sl)

        @pl.when(ch >= 2)
        def _():
            _scatter_add_wait(edv.at[sl], den_sh, dscv.at[sl], semd[sl],
                              nsub=1)
            _scatter_add_wait(wvv.at[sl], tn_sh, dscv.at[sl], semt[sl],
                              nsub=1)

        _copy_idx(dstv.at[sl], dscv.at[sl], nsub=1)

        def eblock(j, _):
            row = idiv + 2 * j
            a16 = plsc.load_gather(attnv.at[sl], [row, imod])
            m16 = plsc.load_gather(mrows.at[sl], [row, imod])
            plsc.store_scatter(edv.at[sl], [row, imod], jnp.exp(a16 - m16))
            return 0

        pl.loop(0, _CH_C * N_HEADS // 16, init_carry=0)(eblock)
        _scatter_add_issue(edv.at[sl], den_sh, dscv.at[sl], semd[sl], nsub=1)

        def wblock(h, _):
            hcol = jnp.full((16,), h, jnp.int32)
            es = [plsc.load_gather(edv.at[sl], [eids[g], hcol])
                  for g in range(8)]
            for d in range(D_K):
                col = jnp.full((16,), h * 8 + d, jnp.int32)
                for g in range(8):
                    v16 = plsc.load_gather(v2rows.at[sl], [eids[g], col])
                    plsc.store_scatter(wvv.at[sl], [eids[g], col],
                                       es[g] * v16)
            return 0

        pl.loop(0, N_HEADS, init_carry=0)(wblock)
        _scatter_add_issue(wvv.at[sl], tn_sh, dscv.at[sl], semt[sl], nsub=1)

        @pl.when(ch + 2 < _CPW_C)
        def _():
            in_issue(ch + 2, sl)

    def pair(p, _):
        half(2 * p, 0, 1)
        half(2 * p + 1, 1, 0)
        return 0

    pl.loop(0, _NPAIR_C, init_carry=0)(pair)
    for sl in range(2):
        _scatter_add_wait(edv.at[sl], den_sh, dscv.at[sl], semd[sl], nsub=1)
        _scatter_add_wait(wvv.at[sl], tn_sh, dscv.at[sl], semt[sl], nsub=1)
    plsc.subcore_barrier()
    for off, nr in ((0, 128), (128, 128), (256, 128), (384, 128), (512, 116)):
        rr = pl.ds(r0 + off, nr)
        bb = pl.ds(0, nr)
        pltpu.sync_copy(den_sh.at[rr], edv.at[0].at[bb])
        pltpu.sync_copy(edv.at[0].at[bb], den_out.at[c, rr])
        pltpu.sync_copy(tn_sh.at[rr], wvv.at[0].at[bb])
        pltpu.sync_copy(wvv.at[0].at[bb], tn_out.at[c, rr])


def _scc(attn, src2, dst2, et2, m, v2f, z8, z64):
    mesh = plsc.VectorSubcoreMesh(core_axis_name="c", subcore_axis_name="s")
    f = pl.kernel(
        _scc_body,
        out_type=[
            jax.ShapeDtypeStruct((2, NA, N_HEADS), jnp.float32),
            jax.ShapeDtypeStruct((2, NA, OUT_DIM), jnp.float32),
        ],
        mesh=mesh,
        compiler_params=_SC_PARAMS,
        scratch_types=[
            pltpu.VMEM((2, 1, 128), jnp.int32),
            pltpu.VMEM((2, 1, 128), jnp.int32),
            pltpu.VMEM((2, 1, 128), jnp.int32),
            pltpu.VMEM((2, 1, 128), jnp.int32),
            pltpu.VMEM((2, 1, 128), jnp.int32),
            pltpu.VMEM((2, _CH_C, N_HEADS), jnp.float32),
            pltpu.VMEM((2, _CH_C, N_HEADS), jnp.float32),
            pltpu.VMEM((2, _CH_C, OUT_DIM), jnp.float32),
            pltpu.VMEM((2, _CH_C, N_HEADS), jnp.float32),
            pltpu.VMEM((2, _CH_C, OUT_DIM), jnp.float32),
            pltpu.VMEM_SHARED((NA, N_HEADS), jnp.float32),
            pltpu.VMEM_SHARED((NA, OUT_DIM), jnp.float32),
            [pltpu.SemaphoreType.DMA, pltpu.SemaphoreType.DMA],
            [pltpu.SemaphoreType.DMA, pltpu.SemaphoreType.DMA],
            [pltpu.SemaphoreType.DMA, pltpu.SemaphoreType.DMA],
            [pltpu.SemaphoreType.DMA, pltpu.SemaphoreType.DMA],
        ],
    )
    return f(attn, src2, dst2, et2, m, v2f, z8, z64)


def _tc2_body(tn_ref, den_ref, nt_ref, a2_ref, p8_ref, out_ref):
    d = den_ref[0] + den_ref[1]
    d64 = jnp.dot(d, p8_ref[...], preferred_element_type=jnp.float32)
    tn = tn_ref[0] + tn_ref[1]
    safe = jnp.where(d64 > 0, d64, 1.0)
    t = jnp.where(d64 > 0, tn / safe, 0.0)
    nt = nt_ref[...]
    acc = jnp.zeros((t.shape[0], OUT_DIM), jnp.float32)
    for ty in range(N_NTYPES):
        acc = acc + jnp.where(
            nt == ty,
            jnp.dot(t, a2_ref[ty], preferred_element_type=jnp.float32), 0.0)
    out_ref[...] = acc


def _tc2(tnp, denp, nt_pad, a2, p8):
    grid = NA // _TCB
    return pl.pallas_call(
        _tc2_body,
        grid=(grid,),
        in_specs=[
            pl.BlockSpec((2, _TCB, OUT_DIM), lambda i: (0, i, 0)),
            pl.BlockSpec((2, _TCB, N_HEADS), lambda i: (0, i, 0)),
            pl.BlockSpec((_TCB, 1), lambda i: (i, 0)),
            pl.BlockSpec((N_NTYPES, OUT_DIM, OUT_DIM), lambda i: (0, 0, 0)),
            pl.BlockSpec((N_HEADS, OUT_DIM), lambda i: (0, 0)),
        ],
        out_specs=pl.BlockSpec((_TCB, OUT_DIM), lambda i: (i, 0)),
        out_shape=jax.ShapeDtypeStruct((NA, OUT_DIM), jnp.float32),
    )(tnp, denp, nt_pad, a2, p8)


def kernel(h, edge_index, ntype, etype, k_weight, q_weight, v_weight,
           a_weight, relation_pri, relation_att, relation_msg, skip):
    N = N_NODES
    f32 = jnp.float32

    h_pad = jnp.pad(h, ((0, NA - N), (0, 0)))
    nt_pad = jnp.pad(ntype.astype(jnp.int32), (0, NA - N)).reshape(NA, 1)
    src = jnp.pad(edge_index[0].astype(jnp.int32), (0, E_PAD - N_EDGES))
    dst = jnp.pad(edge_index[1].astype(jnp.int32), (0, E_PAD - N_EDGES),
                  constant_values=N)
    et = jnp.pad(etype.astype(jnp.int32), (0, E_PAD - N_EDGES))
    src2 = src.reshape(E_PAD // 128, 128)
    dst2 = dst.reshape(E_PAD // 128, 128)
    et2 = et.reshape(E_PAD // 128, 128)

    wcat = jnp.concatenate(
        [w.transpose(1, 0, 2).reshape(IN_DIM, 512)
         for w in (k_weight, q_weight, v_weight)], axis=1)
    scale = relation_pri / math.sqrt(OUT_DIM)
    eye = jnp.eye(N_HEADS, dtype=f32)
    ba = jnp.einsum('rhdf,rh,hg->rhdgf', relation_att, scale, eye)
    ba = ba.reshape(N_RELS, OUT_DIM, OUT_DIM)
    bm = jnp.einsum('rhdf,hg->rhdgf', relation_msg, eye)
    bm = bm.reshape(N_RELS, OUT_DIM, OUT_DIM)
    bacat = ba.transpose(1, 0, 2).reshape(OUT_DIM, 512)
    bmcat = bm.transpose(1, 0, 2).reshape(OUT_DIM, 512)

    qn, k2, v2 = _tc0(h_pad, nt_pad, wcat, bacat, bmcat)
    k2f = k2.reshape(N_RELS * NA, OUT_DIM)
    v2f = v2.reshape(N_RELS * NA, OUT_DIM)

    attn, tmax = _sca(k2f, qn, src2, dst2, et2)

    gm = jnp.max(tmax)
    lam = jnp.maximum(jnp.float32(4.0), gm / 75.0)
    invlam = jnp.full((16,), 1.0, f32) / lam
    z8 = jnp.zeros((NA, N_HEADS), f32)
    z64 = jnp.zeros((NA, OUT_DIM), f32)

    s1p = _scb(attn, dst2, invlam, z8)
    m = _tc1(s1p, lam.reshape(1, 1))

    denp, tnp = _scc(attn, src2, dst2, et2, m, v2f, z8, z64)

    alpha = jax.nn.sigmoid(skip)
    alpha64 = jnp.repeat(alpha, D_K, axis=1)
    a2 = a_weight * alpha64[:, None, :]
    p8 = jnp.repeat(eye, D_K, axis=1)

    out = _tc2(tnp, denp, nt_pad, a2, p8)
    return out[:N]
